# Initial kernel scaffold; baseline (speedup 1.0000x reference)
import jax
import jax.numpy as jnp
from jax.experimental import pallas as pl


def kernel(E_u_0, E_i_0, adj_vals, ut, vt, u_mul_s, v_mul_s, adj_rows, adj_cols, users, pos_items, neg_items):
    raise NotImplementedError("write your pallas kernel here")


# speedup vs baseline: 1.0390x; 1.0390x over previous
import functools

import jax
import jax.numpy as jnp
from jax.experimental import pallas as pl
from jax.experimental.pallas import tpu as pltpu

N_USERS = 50000
N_ITEMS = 50000
DIM = 64
NNZ = 1600000
Q = 5
L = 2
BATCH = 1024
TEMP = 0.2
T2 = 0.1
T3 = 1.0
W1 = 0.2
L2 = 1e-05


def _normalize(x, eps=1e-12):
    n = jnp.sqrt(jnp.sum(x * x, axis=-1, keepdims=True))
    return x / jnp.maximum(n, eps)


def _ypred_kernel(u_ref, p_ref, o_ref):
    o_ref[...] = jnp.dot(u_ref[...], p_ref[...].T,
                         preferred_element_type=jnp.float32)


def _spmm(vals, rows, cols, X, n_out):
    return jax.ops.segment_sum(vals[:, None] * X[cols], rows, num_segments=n_out)


def kernel(E_u_0, E_i_0, adj_vals, ut, vt, u_mul_s, v_mul_s, adj_rows, adj_cols, users, pos_items, neg_items):
    neg = neg_items[:, 0]
    iids = jnp.concatenate([pos_items, neg], axis=0)

    S_u1 = E_u_0 + _spmm(adj_vals, adj_rows, adj_cols, E_i_0, N_USERS)
    S_i1 = E_i_0 + _spmm(adj_vals, adj_cols, adj_rows, E_u_0, N_ITEMS)
    E_u = E_u_0 + _spmm(adj_vals, adj_rows, adj_cols, S_i1, N_USERS)
    E_i = E_i_0 + _spmm(adj_vals, adj_cols, adj_rows, S_u1, N_ITEMS)

    vt_ei = vt @ S_i1
    ut_eu = ut @ S_u1
    G_u_b = E_u_0[users] + u_mul_s[users] @ vt_ei
    G_i_b = E_i_0[iids] + v_mul_s[iids] @ ut_eu

    neg_score = jnp.log(jnp.exp(G_u_b @ E_u.T / TEMP).sum(1) + 1e-08).mean()
    neg_score = neg_score + jnp.log(jnp.exp(G_i_b @ E_i.T / TEMP).sum(1) + 1e-08).mean()
    pos_score = jnp.clip((G_u_b * E_u[users]).sum(1) / TEMP, -5.0, 5.0).mean() \
        + jnp.clip((G_i_b * E_i[iids]).sum(1) / TEMP, -5.0, 5.0).mean()
    loss_s = -pos_score + neg_score

    u_e = _normalize(E_u[users])
    pos_e = _normalize(E_i[pos_items])
    y_pred = pl.pallas_call(
        _ypred_kernel,
        out_shape=jax.ShapeDtypeStruct((BATCH, BATCH), jnp.float32),
    )(u_e, pos_e)
    idx = jnp.arange(BATCH)
    diag = jnp.diagonal(y_pred)
    col0 = y_pred[:, 0]
    y2 = y_pred.at[idx, idx].set(col0)
    y2 = y2.at[idx, 0].set(diag)
    pos_logits = y2[:, 0] / T2
    loss_r = jnp.mean(-pos_logits + jax.nn.logsumexp(y2 / T3, axis=1))
    loss_reg = L2 * (jnp.sum(E_u_0 ** 2) + jnp.sum(E_i_0 ** 2))
    loss = loss_r + W1 * loss_s + loss_reg
    return (loss, loss_r, W1 * loss_s)


# speedup vs baseline: 6.2061x; 5.9732x over previous
import functools

import jax
import jax.numpy as jnp
from jax import lax
from jax.experimental import pallas as pl
from jax.experimental.pallas import tpu as pltpu
from jax.experimental.pallas import tpu_sc as plsc

N_USERS = 50000
N_ITEMS = 50000
DIM = 64
NNZ = 1600000
Q = 5
BATCH = 1024
TEMP = 0.2
T2 = 0.1
T3 = 1.0
W1 = 0.2
L2 = 1e-05

HALF = DIM // 2
NS = 16
CH = 512
JB = CH // 128
NNZ_PAD = 1605632
N_CHUNKS = NNZ_PAD // CH
CHUNKS_PER_TILE = N_CHUNKS // NS
N_ROWS_PAD = 50048
ROWS_PT = N_ROWS_PAD // NS


def _spmm_body(tbl0, tbl1, init0, init1, rows3, cols3, vals2,
               out0, out1, acc, rows_v, cols_v, vals_v, gbuf, gsem, ssem):
    c = lax.axis_index("c")
    s = lax.axis_index("s")
    r0 = pl.multiple_of(s * ROWS_PT, 8)

    @pl.when(c == 0)
    def _():
        pltpu.sync_copy(init0.at[pl.ds(r0, ROWS_PT)], acc.at[pl.ds(r0, ROWS_PT)])

    @pl.when(c == 1)
    def _():
        pltpu.sync_copy(init1.at[pl.ds(r0, ROWS_PT)], acc.at[pl.ds(r0, ROWS_PT)])

    plsc.subcore_barrier()

    def chunk_body(i, carry):
        ci = s * CHUNKS_PER_TILE + i
        pltpu.sync_copy(rows3.at[ci], rows_v)
        pltpu.sync_copy(cols3.at[ci], cols_v)
        pltpu.sync_copy(vals2.at[ci], vals_v)

        @pl.when(c == 0)
        def _():
            ds = [pltpu.async_copy(tbl0.at[cols_v.at[j]],
                                   gbuf.at[pl.ds(j * 128, 128)], gsem)
                  for j in range(JB)]
            for d in ds:
                d.wait()

        @pl.when(c == 1)
        def _():
            ds = [pltpu.async_copy(tbl1.at[cols_v.at[j]],
                                   gbuf.at[pl.ds(j * 128, 128)], gsem)
                  for j in range(JB)]
            for d in ds:
                d.wait()

        def scale16(g, carry2):
            e0 = g * 16
            v16 = vals_v[pl.ds(e0, 16)]
            for l in range(16):
                v = v16[l]
                e = e0 + l
                x0 = gbuf[e, pl.ds(0, 16)]
                x1 = gbuf[e, pl.ds(16, 16)]
                gbuf[e, pl.ds(0, 16)] = x0 * v
                gbuf[e, pl.ds(16, 16)] = x1 * v
            return carry2

        lax.fori_loop(0, CH // 16, scale16, 0)

        sd = [pltpu.async_copy(gbuf.at[pl.ds(j * 128, 128)],
                               acc.at[rows_v.at[j]], ssem, add=True)
              for j in range(JB)]
        for d in sd:
            d.wait()
        return carry

    lax.fori_loop(0, CHUNKS_PER_TILE, chunk_body, 0)
    plsc.subcore_barrier()

    @pl.when(c == 0)
    def _():
        pltpu.sync_copy(acc.at[pl.ds(r0, ROWS_PT)], out0.at[pl.ds(r0, ROWS_PT)])

    @pl.when(c == 1)
    def _():
        pltpu.sync_copy(acc.at[pl.ds(r0, ROWS_PT)], out1.at[pl.ds(r0, ROWS_PT)])


@jax.jit
def _spmm(tbl0, tbl1, init0, init1, rows3, cols3, vals2):
    mesh = plsc.VectorSubcoreMesh(core_axis_name="c", subcore_axis_name="s")
    f = pl.kernel(
        _spmm_body,
        out_type=(jax.ShapeDtypeStruct((N_ROWS_PAD, HALF), jnp.float32),
                  jax.ShapeDtypeStruct((N_ROWS_PAD, HALF), jnp.float32)),
        mesh=mesh,
        scratch_types=[
            pltpu.VMEM_SHARED((N_ROWS_PAD, HALF), jnp.float32),
            pltpu.VMEM((JB, 128), jnp.int32),
            pltpu.VMEM((JB, 128), jnp.int32),
            pltpu.VMEM((CH,), jnp.float32),
            pltpu.VMEM((CH, HALF), jnp.float32),
            pltpu.SemaphoreType.DMA,
            pltpu.SemaphoreType.DMA,
        ],
        compiler_params=pltpu.CompilerParams(use_tc_tiling_on_sc=False),
    )
    return f(tbl0, tbl1, init0, init1, rows3, cols3, vals2)


def _normalize(x, eps=1e-12):
    n = jnp.sqrt(jnp.sum(x * x, axis=-1, keepdims=True))
    return x / jnp.maximum(n, eps)


def _ypred_kernel(u_ref, p_ref, o_ref):
    o_ref[...] = jnp.dot(u_ref[...], p_ref[...].T,
                         preferred_element_type=jnp.float32)


def kernel(E_u_0, E_i_0, adj_vals, ut, vt, u_mul_s, v_mul_s, adj_rows, adj_cols, users, pos_items, neg_items):
    neg = neg_items[:, 0]
    iids = jnp.concatenate([pos_items, neg], axis=0)

    pad = NNZ_PAD - NNZ
    rows_p = jnp.concatenate([adj_rows, jnp.zeros((pad,), jnp.int32)])
    cols_p = jnp.concatenate([adj_cols, jnp.zeros((pad,), jnp.int32)])
    vals_p = jnp.concatenate([adj_vals, jnp.zeros((pad,), jnp.float32)])
    rows3 = rows_p.reshape(N_CHUNKS, JB, 128)
    cols3 = cols_p.reshape(N_CHUNKS, JB, 128)
    vals2 = vals_p.reshape(N_CHUNKS, CH)

    rpad = N_ROWS_PAD - N_USERS
    Eu0p = jnp.concatenate([E_u_0, jnp.zeros((rpad, DIM), jnp.float32)])
    Ei0p = jnp.concatenate([E_i_0, jnp.zeros((rpad, DIM), jnp.float32)])
    Eu0a, Eu0b = Eu0p[:, :HALF], Eu0p[:, HALF:]
    Ei0a, Ei0b = Ei0p[:, :HALF], Ei0p[:, HALF:]

    su1a, su1b = _spmm(Ei0a[:N_ITEMS], Ei0b[:N_ITEMS], Eu0a, Eu0b, rows3, cols3, vals2)
    si1a, si1b = _spmm(Eu0a[:N_USERS], Eu0b[:N_USERS], Ei0a, Ei0b, cols3, rows3, vals2)
    eua, eub = _spmm(si1a[:N_ITEMS], si1b[:N_ITEMS], Eu0a, Eu0b, rows3, cols3, vals2)
    eia, eib = _spmm(su1a[:N_USERS], su1b[:N_USERS], Ei0a, Ei0b, cols3, rows3, vals2)

    E_u = jnp.concatenate([eua[:N_USERS], eub[:N_USERS]], axis=1)
    E_i = jnp.concatenate([eia[:N_ITEMS], eib[:N_ITEMS]], axis=1)
    S_u1 = jnp.concatenate([su1a[:N_USERS], su1b[:N_USERS]], axis=1)
    S_i1 = jnp.concatenate([si1a[:N_ITEMS], si1b[:N_ITEMS]], axis=1)

    vt_ei = vt @ S_i1
    ut_eu = ut @ S_u1
    G_u_b = E_u_0[users] + u_mul_s[users] @ vt_ei
    G_i_b = E_i_0[iids] + v_mul_s[iids] @ ut_eu

    neg_score = jnp.log(jnp.exp(G_u_b @ E_u.T / TEMP).sum(1) + 1e-08).mean()
    neg_score = neg_score + jnp.log(jnp.exp(G_i_b @ E_i.T / TEMP).sum(1) + 1e-08).mean()
    pos_score = jnp.clip((G_u_b * E_u[users]).sum(1) / TEMP, -5.0, 5.0).mean() \
        + jnp.clip((G_i_b * E_i[iids]).sum(1) / TEMP, -5.0, 5.0).mean()
    loss_s = -pos_score + neg_score

    u_e = _normalize(E_u[users])
    pos_e = _normalize(E_i[pos_items])
    y_pred = pl.pallas_call(
        _ypred_kernel,
        out_shape=jax.ShapeDtypeStruct((BATCH, BATCH), jnp.float32),
    )(u_e, pos_e)
    idx = jnp.arange(BATCH)
    diag = jnp.diagonal(y_pred)
    col0 = y_pred[:, 0]
    y2 = y_pred.at[idx, idx].set(col0)
    y2 = y2.at[idx, 0].set(diag)
    pos_logits = y2[:, 0] / T2
    loss_r = jnp.mean(-pos_logits + jax.nn.logsumexp(y2 / T3, axis=1))
    loss_reg = L2 * (jnp.sum(E_u_0 ** 2) + jnp.sum(E_i_0 ** 2))
    loss = loss_r + W1 * loss_s + loss_reg
    return (loss, loss_r, W1 * loss_s)


# speedup vs baseline: 8.0690x; 1.3002x over previous
import functools

import jax
import jax.numpy as jnp
from jax import lax
from jax.experimental import pallas as pl
from jax.experimental.pallas import tpu as pltpu
from jax.experimental.pallas import tpu_sc as plsc

N_USERS = 50000
N_ITEMS = 50000
DIM = 64
NNZ = 1600000
Q = 5
BATCH = 1024
TEMP = 0.2
T2 = 0.1
T3 = 1.0
W1 = 0.2
L2 = 1e-05

HALF = DIM // 2
NS = 16
CH = 384
JB = CH // 128
NNZ_PAD = 1609728
N_CHUNKS = NNZ_PAD // CH
CHUNKS_PER_TILE = N_CHUNKS // NS
NPAIRS = CHUNKS_PER_TILE // 2
N_ROWS_PAD = 50048
ROWS_PT = N_ROWS_PAD // NS


def _spmm_body(tbl0, tbl1, init0, init1, ids2, vals2,
               out0, out1, acc, idsA, idsB, valsA, valsB, gbufA, gbufB,
               gsA, gsB, ssA, ssB):
    c = lax.axis_index("c")
    s = lax.axis_index("s")
    r0 = pl.multiple_of(s * ROWS_PT, 8)

    @pl.when(c == 0)
    def _():
        pltpu.sync_copy(init0.at[pl.ds(r0, ROWS_PT)], acc.at[pl.ds(r0, ROWS_PT)])

    @pl.when(c == 1)
    def _():
        pltpu.sync_copy(init1.at[pl.ds(r0, ROWS_PT)], acc.at[pl.ds(r0, ROWS_PT)])

    plsc.subcore_barrier()

    base = s * CHUNKS_PER_TILE

    def load_ids(ci, ids, vals):
        pltpu.sync_copy(ids2.at[ci], ids)
        pltpu.sync_copy(vals2.at[ci], vals)

    def fire_gather(ids, gbuf, sem):
        @pl.when(c == 0)
        def _():
            for j in range(JB):
                pltpu.async_copy(tbl0.at[ids.at[JB + j]],
                                 gbuf.at[pl.ds(j * 128, 128)], sem)

        @pl.when(c == 1)
        def _():
            for j in range(JB):
                pltpu.async_copy(tbl1.at[ids.at[JB + j]],
                                 gbuf.at[pl.ds(j * 128, 128)], sem)

    def drain_gather(ids, gbuf, sem):
        for j in range(JB):
            pltpu.make_async_copy(tbl0.at[ids.at[JB + j]],
                                  gbuf.at[pl.ds(j * 128, 128)], sem).wait()

    def fire_scatter(ids, gbuf, sem):
        for j in range(JB):
            pltpu.async_copy(gbuf.at[pl.ds(j * 128, 128)],
                             acc.at[ids.at[j]], sem, add=True)

    def drain_scatter(ids, gbuf, sem):
        for j in range(JB):
            pltpu.make_async_copy(gbuf.at[pl.ds(j * 128, 128)],
                                  acc.at[ids.at[j]], sem).wait()

    def scale(vals, gbuf):
        def scale16(g, carry2):
            e0 = g * 16
            v16 = vals[pl.ds(e0, 16)]
            for l in range(16):
                v = v16[l]
                e = e0 + l
                x0 = gbuf[e, pl.ds(0, 16)]
                x1 = gbuf[e, pl.ds(16, 16)]
                gbuf[e, pl.ds(0, 16)] = x0 * v
                gbuf[e, pl.ds(16, 16)] = x1 * v
            return carry2

        lax.fori_loop(0, CH // 16, scale16, 0)

    load_ids(base, idsA, valsA)
    fire_gather(idsA, gbufA, gsA)

    def pair_body(i, carry):
        cA = base + 2 * i
        load_ids(cA + 1, idsB, valsB)
        fire_gather(idsB, gbufB, gsB)
        drain_gather(idsA, gbufA, gsA)
        scale(valsA, gbufA)
        fire_scatter(idsA, gbufA, ssA)
        drain_gather(idsB, gbufB, gsB)
        scale(valsB, gbufB)
        fire_scatter(idsB, gbufB, ssB)
        drain_scatter(idsA, gbufA, ssA)

        @pl.when(i + 1 < NPAIRS)
        def _():
            load_ids(cA + 2, idsA, valsA)
            fire_gather(idsA, gbufA, gsA)

        drain_scatter(idsB, gbufB, ssB)
        return carry

    lax.fori_loop(0, NPAIRS, pair_body, 0)
    plsc.subcore_barrier()

    @pl.when(c == 0)
    def _():
        pltpu.sync_copy(acc.at[pl.ds(r0, ROWS_PT)], out0.at[pl.ds(r0, ROWS_PT)])

    @pl.when(c == 1)
    def _():
        pltpu.sync_copy(acc.at[pl.ds(r0, ROWS_PT)], out1.at[pl.ds(r0, ROWS_PT)])


@jax.jit
def _spmm(tbl0, tbl1, init0, init1, ids2, vals2):
    mesh = plsc.VectorSubcoreMesh(core_axis_name="c", subcore_axis_name="s")
    f = pl.kernel(
        _spmm_body,
        out_type=(jax.ShapeDtypeStruct((N_ROWS_PAD, HALF), jnp.float32),
                  jax.ShapeDtypeStruct((N_ROWS_PAD, HALF), jnp.float32)),
        mesh=mesh,
        scratch_types=[
            pltpu.VMEM_SHARED((N_ROWS_PAD, HALF), jnp.float32),
            pltpu.VMEM((2 * JB, 128), jnp.int32),
            pltpu.VMEM((2 * JB, 128), jnp.int32),
            pltpu.VMEM((CH,), jnp.float32),
            pltpu.VMEM((CH,), jnp.float32),
            pltpu.VMEM((CH, HALF), jnp.float32),
            pltpu.VMEM((CH, HALF), jnp.float32),
            pltpu.SemaphoreType.DMA,
            pltpu.SemaphoreType.DMA,
            pltpu.SemaphoreType.DMA,
            pltpu.SemaphoreType.DMA,
        ],
        compiler_params=pltpu.CompilerParams(use_tc_tiling_on_sc=False),
    )
    return f(tbl0, tbl1, init0, init1, ids2, vals2)


def _normalize(x, eps=1e-12):
    n = jnp.sqrt(jnp.sum(x * x, axis=-1, keepdims=True))
    return x / jnp.maximum(n, eps)


def _ypred_kernel(u_ref, p_ref, o_ref):
    o_ref[...] = jnp.dot(u_ref[...], p_ref[...].T,
                         preferred_element_type=jnp.float32)


def kernel(E_u_0, E_i_0, adj_vals, ut, vt, u_mul_s, v_mul_s, adj_rows, adj_cols, users, pos_items, neg_items):
    neg = neg_items[:, 0]
    iids = jnp.concatenate([pos_items, neg], axis=0)

    pad = NNZ_PAD - NNZ
    rows_p = jnp.concatenate([adj_rows, jnp.zeros((pad,), jnp.int32)])
    cols_p = jnp.concatenate([adj_cols, jnp.zeros((pad,), jnp.int32)])
    vals_p = jnp.concatenate([adj_vals, jnp.zeros((pad,), jnp.float32)])
    ids_uv = jnp.concatenate([rows_p.reshape(N_CHUNKS, CH),
                              cols_p.reshape(N_CHUNKS, CH)],
                             axis=1).reshape(N_CHUNKS, 2 * JB, 128)
    ids_iv = jnp.concatenate([cols_p.reshape(N_CHUNKS, CH),
                              rows_p.reshape(N_CHUNKS, CH)],
                             axis=1).reshape(N_CHUNKS, 2 * JB, 128)
    vals2 = vals_p.reshape(N_CHUNKS, CH)

    rpad = N_ROWS_PAD - N_USERS
    Eu0p = jnp.concatenate([E_u_0, jnp.zeros((rpad, DIM), jnp.float32)])
    Ei0p = jnp.concatenate([E_i_0, jnp.zeros((rpad, DIM), jnp.float32)])
    Eu0a, Eu0b = Eu0p[:, :HALF], Eu0p[:, HALF:]
    Ei0a, Ei0b = Ei0p[:, :HALF], Ei0p[:, HALF:]

    su1a, su1b = _spmm(Ei0a[:N_ITEMS], Ei0b[:N_ITEMS], Eu0a, Eu0b, ids_uv, vals2)
    si1a, si1b = _spmm(Eu0a[:N_USERS], Eu0b[:N_USERS], Ei0a, Ei0b, ids_iv, vals2)
    eua, eub = _spmm(si1a[:N_ITEMS], si1b[:N_ITEMS], Eu0a, Eu0b, ids_uv, vals2)
    eia, eib = _spmm(su1a[:N_USERS], su1b[:N_USERS], Ei0a, Ei0b, ids_iv, vals2)

    E_u = jnp.concatenate([eua[:N_USERS], eub[:N_USERS]], axis=1)
    E_i = jnp.concatenate([eia[:N_ITEMS], eib[:N_ITEMS]], axis=1)
    S_u1 = jnp.concatenate([su1a[:N_USERS], su1b[:N_USERS]], axis=1)
    S_i1 = jnp.concatenate([si1a[:N_ITEMS], si1b[:N_ITEMS]], axis=1)

    vt_ei = vt @ S_i1
    ut_eu = ut @ S_u1
    G_u_b = E_u_0[users] + u_mul_s[users] @ vt_ei
    G_i_b = E_i_0[iids] + v_mul_s[iids] @ ut_eu

    neg_score = jnp.log(jnp.exp(G_u_b @ E_u.T / TEMP).sum(1) + 1e-08).mean()
    neg_score = neg_score + jnp.log(jnp.exp(G_i_b @ E_i.T / TEMP).sum(1) + 1e-08).mean()
    pos_score = jnp.clip((G_u_b * E_u[users]).sum(1) / TEMP, -5.0, 5.0).mean() \
        + jnp.clip((G_i_b * E_i[iids]).sum(1) / TEMP, -5.0, 5.0).mean()
    loss_s = -pos_score + neg_score

    u_e = _normalize(E_u[users])
    pos_e = _normalize(E_i[pos_items])
    y_pred = pl.pallas_call(
        _ypred_kernel,
        out_shape=jax.ShapeDtypeStruct((BATCH, BATCH), jnp.float32),
    )(u_e, pos_e)
    idx = jnp.arange(BATCH)
    diag = jnp.diagonal(y_pred)
    col0 = y_pred[:, 0]
    y2 = y_pred.at[idx, idx].set(col0)
    y2 = y2.at[idx, 0].set(diag)
    pos_logits = y2[:, 0] / T2
    loss_r = jnp.mean(-pos_logits + jax.nn.logsumexp(y2 / T3, axis=1))
    loss_reg = L2 * (jnp.sum(E_u_0 ** 2) + jnp.sum(E_i_0 ** 2))
    loss = loss_r + W1 * loss_s + loss_reg
    return (loss, loss_r, W1 * loss_s)


# speedup vs baseline: 10.7590x; 1.3334x over previous
import functools

import jax
import jax.numpy as jnp
from jax import lax
from jax.experimental import pallas as pl
from jax.experimental.pallas import tpu as pltpu
from jax.experimental.pallas import tpu_sc as plsc

N_USERS = 50000
N_ITEMS = 50000
DIM = 64
NNZ = 1600000
Q = 5
BATCH = 1024
TEMP = 0.2
T2 = 0.1
T3 = 1.0
W1 = 0.2
L2 = 1e-05

HALF = DIM // 2
NS = 16
CH = 256
JB = CH // 128
NNZ_PAD = 1609728
N_CHUNKS = NNZ_PAD // CH
CHUNKS_PER_TILE = N_CHUNKS // NS
NTRIP = CHUNKS_PER_TILE // 3
N_ROWS_PAD = 50048
ROWS_PT = N_ROWS_PAD // NS


def _spmm_body(tbl0, tbl1, init0, init1, ids2, vals2,
               out0, out1, acc,
               ids_r, vals_r, srows_r, gbuf_r,
               gsem_r, ssem_r, isem_r):
    c = lax.axis_index("c")
    s = lax.axis_index("s")
    r0 = pl.multiple_of(s * ROWS_PT, 8)

    @pl.when(c == 0)
    def _():
        pltpu.sync_copy(init0.at[pl.ds(r0, ROWS_PT)], acc.at[pl.ds(r0, ROWS_PT)])

    @pl.when(c == 1)
    def _():
        pltpu.sync_copy(init1.at[pl.ds(r0, ROWS_PT)], acc.at[pl.ds(r0, ROWS_PT)])

    plsc.subcore_barrier()

    base = s * CHUNKS_PER_TILE
    NCHT = CHUNKS_PER_TILE

    def prefetch_ids(ci, r):
        pltpu.async_copy(ids2.at[ci], ids_r[r], isem_r[r])
        pltpu.async_copy(vals2.at[ci], vals_r[r], isem_r[r])

    def wait_ids(r):
        pltpu.make_async_copy(ids2.at[0], ids_r[r], isem_r[r]).wait()
        pltpu.make_async_copy(vals2.at[0], vals_r[r], isem_r[r]).wait()

    def fire_gather(r):
        @pl.when(c == 0)
        def _():
            for j in range(JB):
                pltpu.async_copy(tbl0.at[ids_r[r].at[JB + j]],
                                 gbuf_r[r].at[pl.ds(j * 128, 128)], gsem_r[r])

        @pl.when(c == 1)
        def _():
            for j in range(JB):
                pltpu.async_copy(tbl1.at[ids_r[r].at[JB + j]],
                                 gbuf_r[r].at[pl.ds(j * 128, 128)], gsem_r[r])

    def drain_gather(r):
        for j in range(JB):
            pltpu.make_async_copy(tbl0.at[ids_r[r].at[JB + j]],
                                  gbuf_r[r].at[pl.ds(j * 128, 128)], gsem_r[r]).wait()

    def copy_rows(r):
        for t in range(8 * JB):
            srows_r[r][t // 8, pl.ds((t % 8) * 16, 16)] = (
                ids_r[r][t // 8, pl.ds((t % 8) * 16, 16)])

    def fire_scatter(r):
        for j in range(JB):
            pltpu.async_copy(gbuf_r[r].at[pl.ds(j * 128, 128)],
                             acc.at[srows_r[r].at[j]], ssem_r[r], add=True)

    def drain_scatter(r):
        for j in range(JB):
            pltpu.make_async_copy(gbuf_r[r].at[pl.ds(j * 128, 128)],
                                  acc.at[srows_r[r].at[j]], ssem_r[r]).wait()

    def scale(r):
        gbuf = gbuf_r[r]
        vals = vals_r[r]

        def scale16(g, carry2):
            e0 = g * 16
            v16 = vals[pl.ds(e0, 16)]
            for l in range(16):
                v = v16[l]
                e = e0 + l
                x0 = gbuf[e, pl.ds(0, 16)]
                x1 = gbuf[e, pl.ds(16, 16)]
                gbuf[e, pl.ds(0, 16)] = x0 * v
                gbuf[e, pl.ds(16, 16)] = x1 * v
            return carry2

        lax.fori_loop(0, CH // 16, scale16, 0)

    pltpu.sync_copy(ids2.at[base], ids_r[0])
    pltpu.sync_copy(vals2.at[base], vals_r[0])
    pltpu.sync_copy(ids2.at[base + 1], ids_r[1])
    pltpu.sync_copy(vals2.at[base + 1], vals_r[1])
    fire_gather(0)
    fire_gather(1)
    prefetch_ids(base + 2, 2)

    def trip_body(i, carry):
        for r in range(3):
            j = 3 * i + r
            drain_gather(r)
            copy_rows(r)
            scale(r)
            fire_scatter(r)

            @pl.when(j + 3 < NCHT)
            def _():
                prefetch_ids(base + j + 3, r)

            @pl.when(j + 2 < NCHT)
            def _():
                wait_ids((r + 2) % 3)

            @pl.when(j > 0)
            def _():
                drain_scatter((r + 2) % 3)

            @pl.when(j + 2 < NCHT)
            def _():
                fire_gather((r + 2) % 3)
        return carry

    lax.fori_loop(0, NTRIP, trip_body, 0)
    drain_scatter((CHUNKS_PER_TILE - 1) % 3)
    plsc.subcore_barrier()

    @pl.when(c == 0)
    def _():
        pltpu.sync_copy(acc.at[pl.ds(r0, ROWS_PT)], out0.at[pl.ds(r0, ROWS_PT)])

    @pl.when(c == 1)
    def _():
        pltpu.sync_copy(acc.at[pl.ds(r0, ROWS_PT)], out1.at[pl.ds(r0, ROWS_PT)])


@jax.jit
def _spmm(tbl0, tbl1, init0, init1, ids2, vals2):
    mesh = plsc.VectorSubcoreMesh(core_axis_name="c", subcore_axis_name="s")
    f = pl.kernel(
        _spmm_body,
        out_type=(jax.ShapeDtypeStruct((N_ROWS_PAD, HALF), jnp.float32),
                  jax.ShapeDtypeStruct((N_ROWS_PAD, HALF), jnp.float32)),
        mesh=mesh,
        scratch_types=[
            pltpu.VMEM_SHARED((N_ROWS_PAD, HALF), jnp.float32),
            [pltpu.VMEM((2 * JB, 128), jnp.int32) for _ in range(3)],
            [pltpu.VMEM((CH,), jnp.float32) for _ in range(3)],
            [pltpu.VMEM((JB, 128), jnp.int32) for _ in range(3)],
            [pltpu.VMEM((CH, HALF), jnp.float32) for _ in range(3)],
            [pltpu.SemaphoreType.DMA for _ in range(3)],
            [pltpu.SemaphoreType.DMA for _ in range(3)],
            [pltpu.SemaphoreType.DMA for _ in range(3)],
        ],
        compiler_params=pltpu.CompilerParams(use_tc_tiling_on_sc=False),
    )
    return f(tbl0, tbl1, init0, init1, ids2, vals2)


def _normalize(x, eps=1e-12):
    n = jnp.sqrt(jnp.sum(x * x, axis=-1, keepdims=True))
    return x / jnp.maximum(n, eps)


def _ypred_kernel(u_ref, p_ref, o_ref):
    o_ref[...] = jnp.dot(u_ref[...], p_ref[...].T,
                         preferred_element_type=jnp.float32)


def kernel(E_u_0, E_i_0, adj_vals, ut, vt, u_mul_s, v_mul_s, adj_rows, adj_cols, users, pos_items, neg_items):
    neg = neg_items[:, 0]
    iids = jnp.concatenate([pos_items, neg], axis=0)

    pad = NNZ_PAD - NNZ
    rows_p = jnp.concatenate([adj_rows, jnp.zeros((pad,), jnp.int32)])
    cols_p = jnp.concatenate([adj_cols, jnp.zeros((pad,), jnp.int32)])
    vals_p = jnp.concatenate([adj_vals, jnp.zeros((pad,), jnp.float32)])
    ids_uv = jnp.concatenate([rows_p.reshape(N_CHUNKS, CH),
                              cols_p.reshape(N_CHUNKS, CH)],
                             axis=1).reshape(N_CHUNKS, 2 * JB, 128)
    ids_iv = jnp.concatenate([cols_p.reshape(N_CHUNKS, CH),
                              rows_p.reshape(N_CHUNKS, CH)],
                             axis=1).reshape(N_CHUNKS, 2 * JB, 128)
    vals2 = vals_p.reshape(N_CHUNKS, CH)

    rpad = N_ROWS_PAD - N_USERS
    Eu0p = jnp.concatenate([E_u_0, jnp.zeros((rpad, DIM), jnp.float32)])
    Ei0p = jnp.concatenate([E_i_0, jnp.zeros((rpad, DIM), jnp.float32)])
    Eu0a, Eu0b = Eu0p[:, :HALF], Eu0p[:, HALF:]
    Ei0a, Ei0b = Ei0p[:, :HALF], Ei0p[:, HALF:]

    su1a, su1b = _spmm(Ei0a[:N_ITEMS], Ei0b[:N_ITEMS], Eu0a, Eu0b, ids_uv, vals2)
    si1a, si1b = _spmm(Eu0a[:N_USERS], Eu0b[:N_USERS], Ei0a, Ei0b, ids_iv, vals2)
    eua, eub = _spmm(si1a[:N_ITEMS], si1b[:N_ITEMS], Eu0a, Eu0b, ids_uv, vals2)
    eia, eib = _spmm(su1a[:N_USERS], su1b[:N_USERS], Ei0a, Ei0b, ids_iv, vals2)

    E_u = jnp.concatenate([eua[:N_USERS], eub[:N_USERS]], axis=1)
    E_i = jnp.concatenate([eia[:N_ITEMS], eib[:N_ITEMS]], axis=1)
    S_u1 = jnp.concatenate([su1a[:N_USERS], su1b[:N_USERS]], axis=1)
    S_i1 = jnp.concatenate([si1a[:N_ITEMS], si1b[:N_ITEMS]], axis=1)

    vt_ei = vt @ S_i1
    ut_eu = ut @ S_u1
    G_u_b = E_u_0[users] + u_mul_s[users] @ vt_ei
    G_i_b = E_i_0[iids] + v_mul_s[iids] @ ut_eu

    neg_score = jnp.log(jnp.exp(G_u_b @ E_u.T / TEMP).sum(1) + 1e-08).mean()
    neg_score = neg_score + jnp.log(jnp.exp(G_i_b @ E_i.T / TEMP).sum(1) + 1e-08).mean()
    pos_score = jnp.clip((G_u_b * E_u[users]).sum(1) / TEMP, -5.0, 5.0).mean() \
        + jnp.clip((G_i_b * E_i[iids]).sum(1) / TEMP, -5.0, 5.0).mean()
    loss_s = -pos_score + neg_score

    u_e = _normalize(E_u[users])
    pos_e = _normalize(E_i[pos_items])
    y_pred = pl.pallas_call(
        _ypred_kernel,
        out_shape=jax.ShapeDtypeStruct((BATCH, BATCH), jnp.float32),
    )(u_e, pos_e)
    idx = jnp.arange(BATCH)
    diag = jnp.diagonal(y_pred)
    col0 = y_pred[:, 0]
    y2 = y_pred.at[idx, idx].set(col0)
    y2 = y2.at[idx, 0].set(diag)
    pos_logits = y2[:, 0] / T2
    loss_r = jnp.mean(-pos_logits + jax.nn.logsumexp(y2 / T3, axis=1))
    loss_reg = L2 * (jnp.sum(E_u_0 ** 2) + jnp.sum(E_i_0 ** 2))
    loss = loss_r + W1 * loss_s + loss_reg
    return (loss, loss_r, W1 * loss_s)
